# Initial kernel scaffold; baseline (speedup 1.0000x reference)
import jax
import jax.numpy as jnp
from jax.experimental import pallas as pl


def kernel(x, W_global, W_local, bias, node_ids, neigh_ids, deg):
    raise NotImplementedError("write your pallas kernel here")


# speedup vs baseline: 8.2310x; 8.2310x over previous
import functools

import jax
import jax.numpy as jnp
from jax import lax
from jax.experimental import pallas as pl
from jax.experimental.pallas import tpu as pltpu
from jax.experimental.pallas import tpu_sc as plsc

NUM_CORES = 2
NUM_SUBCORES = 16
NUM_WORKERS = NUM_CORES * NUM_SUBCORES


def _make_segsum_sc(N, E, D, deg):
    n_base = N // NUM_WORKERS
    n_last = N - n_base * (NUM_WORKERS - 1)
    assert n_base % 2 == 0 and n_last % 2 == 0
    max_nodes = n_last

    mesh = plsc.VectorSubcoreMesh(
        core_axis_name="c", subcore_axis_name="s",
        num_cores=NUM_CORES, num_subcores=NUM_SUBCORES)

    @functools.partial(
        pl.kernel,
        out_type=jax.ShapeDtypeStruct((N, D), jnp.float32),
        mesh=mesh,
        scratch_types=[
            pltpu.VMEM((max_nodes * deg,), jnp.int32),
            pltpu.VMEM((deg, D), jnp.float32),
            pltpu.VMEM((deg, D), jnp.float32),
            pltpu.VMEM((max_nodes, D), jnp.float32),
            pltpu.SemaphoreType.DMA,
            pltpu.SemaphoreType.DMA,
        ],
    )
    def segsum(x_hbm, nids_hbm, s_hbm, idx_all, buf0, buf1, out_all,
               sem0, sem1):
        c = lax.axis_index("c")
        s = lax.axis_index("s")
        w = c * NUM_SUBCORES + s
        is_last = w == NUM_WORKERS - 1
        base = w * n_base
        n_w = jnp.where(is_last, n_last, n_base)

        @pl.when(jnp.logical_not(is_last))
        def _():
            pltpu.sync_copy(nids_hbm.at[pl.ds(base * deg, n_base * deg)],
                            idx_all.at[pl.ds(0, n_base * deg)])

        @pl.when(is_last)
        def _():
            pltpu.sync_copy(nids_hbm.at[pl.ds(base * deg, n_last * deg)],
                            idx_all)

        def start_gather(i, buf, sem):
            pltpu.async_copy(
                x_hbm.at[idx_all.at[pl.ds(i * deg, deg)]], buf, sem)

        def wait_gather(buf, sem):
            pltpu.make_async_copy(
                x_hbm.at[idx_all.at[pl.ds(0, deg)]], buf, sem).wait()

        def reduce_into(buf, i):
            for cc in range(D // 16):
                sl = pl.ds(cc * 16, 16)
                acc = buf[0, sl]
                for r in range(1, deg):
                    acc = acc + buf[r, sl]
                out_all[i, sl] = acc

        start_gather(0, buf0, sem0)

        def body(io, carry):
            i = 2 * io
            start_gather(i + 1, buf1, sem1)
            wait_gather(buf0, sem0)
            reduce_into(buf0, i)

            @pl.when(i + 2 < n_w)
            def _():
                start_gather(i + 2, buf0, sem0)

            wait_gather(buf1, sem1)
            reduce_into(buf1, i + 1)
            return carry

        lax.fori_loop(0, n_w // 2, body, 0)

        @pl.when(jnp.logical_not(is_last))
        def _():
            pltpu.sync_copy(out_all.at[pl.ds(0, n_base)],
                            s_hbm.at[pl.ds(base, n_base)])

        @pl.when(is_last)
        def _():
            pltpu.sync_copy(out_all, s_hbm.at[pl.ds(base, n_last)])

    return segsum


def _tc_fuse_body(x_ref, s_ref, wg_ref, wc_ref, b_ref, o_ref):
    xb = x_ref[...]
    sb = s_ref[...]
    dn = (((1,), (1,)), ((), ()))
    o = lax.dot_general(xb, wg_ref[...], dn, preferred_element_type=jnp.float32)
    o = o + lax.dot_general(sb, wc_ref[...], dn,
                            preferred_element_type=jnp.float32)
    o = o + b_ref[...]
    o_ref[...] = jnp.where(o > 0, o, jnp.exp(jnp.minimum(o, 0.0)) - 1.0)


def _tc_fuse(x, S, Wg, Wc, bias2d):
    N, D = x.shape
    DO = Wg.shape[0]
    blk = 1000
    grid = (N // blk,)
    return pl.pallas_call(
        _tc_fuse_body,
        grid=grid,
        in_specs=[
            pl.BlockSpec((blk, D), lambda i: (i, 0)),
            pl.BlockSpec((blk, D), lambda i: (i, 0)),
            pl.BlockSpec((DO, D), lambda i: (0, 0)),
            pl.BlockSpec((DO, D), lambda i: (0, 0)),
            pl.BlockSpec((1, DO), lambda i: (0, 0)),
        ],
        out_specs=pl.BlockSpec((blk, DO), lambda i: (i, 0)),
        out_shape=jax.ShapeDtypeStruct((N, DO), jnp.float32),
    )(x, S, Wg, Wc, bias2d)


def kernel(x, W_global, W_local, bias, node_ids, neigh_ids, deg):
    N, D = x.shape
    E = neigh_ids.shape[0]
    deg_static = E // N
    segsum = _make_segsum_sc(N, E, D, deg_static)
    S = segsum(x, neigh_ids.astype(jnp.int32))
    inv_deg = 1.0 / jnp.asarray(deg, jnp.float32)
    Wc = (W_local + W_global) * inv_deg
    out = _tc_fuse(x, S, W_global, Wc, bias.reshape(1, -1))
    return out


# speedup vs baseline: 9.0206x; 1.0959x over previous
import functools

import jax
import jax.numpy as jnp
from jax import lax
from jax.experimental import pallas as pl
from jax.experimental.pallas import tpu as pltpu
from jax.experimental.pallas import tpu_sc as plsc

NUM_CORES = 2
NUM_SUBCORES = 16
NUM_WORKERS = NUM_CORES * NUM_SUBCORES


def _make_segsum_sc(N, E, D, deg):
    n_base = N // NUM_WORKERS
    n_last = N - n_base * (NUM_WORKERS - 1)
    assert n_base % 2 == 0 and n_last % 2 == 0
    max_nodes = n_last

    mesh = plsc.VectorSubcoreMesh(
        core_axis_name="c", subcore_axis_name="s",
        num_cores=NUM_CORES, num_subcores=NUM_SUBCORES)

    @functools.partial(
        pl.kernel,
        out_type=jax.ShapeDtypeStruct((N, D), jnp.float32),
        mesh=mesh,
        scratch_types=[
            pltpu.VMEM((max_nodes * deg,), jnp.int32),
            pltpu.VMEM((deg, D), jnp.float32),
            pltpu.VMEM((deg, D), jnp.float32),
            pltpu.VMEM((max_nodes, D), jnp.float32),
            pltpu.SemaphoreType.DMA,
            pltpu.SemaphoreType.DMA,
        ],
    )
    def segsum(x_hbm, nids_hbm, s_hbm, idx_all, buf0, buf1, out_all,
               sem0, sem1):
        c = lax.axis_index("c")
        s = lax.axis_index("s")
        w = c * NUM_SUBCORES + s
        is_last = w == NUM_WORKERS - 1
        base = w * n_base
        n_w = jnp.where(is_last, n_last, n_base)

        @pl.when(jnp.logical_not(is_last))
        def _():
            pltpu.sync_copy(nids_hbm.at[pl.ds(base * deg, n_base * deg)],
                            idx_all.at[pl.ds(0, n_base * deg)])

        @pl.when(is_last)
        def _():
            pltpu.sync_copy(nids_hbm.at[pl.ds(base * deg, n_last * deg)],
                            idx_all)

        def start_gather(i, buf, sem):
            pltpu.async_copy(
                x_hbm.at[idx_all.at[pl.ds(i * deg, deg)]], buf, sem)

        def wait_gather(buf, sem):
            pltpu.make_async_copy(
                x_hbm.at[idx_all.at[pl.ds(0, deg)]], buf, sem).wait()

        def reduce_into(buf, i):
            for cc in range(D // 16):
                sl = pl.ds(cc * 16, 16)
                accs = [buf[k, sl] for k in range(4)]
                for r in range(4, deg):
                    k = r % 4
                    accs[k] = accs[k] + buf[r, sl]
                out_all[i, sl] = (accs[0] + accs[1]) + (accs[2] + accs[3])

        start_gather(0, buf0, sem0)

        def body(io, carry):
            i = 2 * io
            start_gather(i + 1, buf1, sem1)
            wait_gather(buf0, sem0)
            reduce_into(buf0, i)

            @pl.when(i + 2 < n_w)
            def _():
                start_gather(i + 2, buf0, sem0)

            wait_gather(buf1, sem1)
            reduce_into(buf1, i + 1)
            return carry

        lax.fori_loop(0, n_w // 2, body, 0)

        @pl.when(jnp.logical_not(is_last))
        def _():
            pltpu.sync_copy(out_all.at[pl.ds(0, n_base)],
                            s_hbm.at[pl.ds(base, n_base)])

        @pl.when(is_last)
        def _():
            pltpu.sync_copy(out_all, s_hbm.at[pl.ds(base, n_last)])

    return segsum


def _tc_fuse_body(x_ref, s_ref, wg_ref, wc_ref, b_ref, o_ref):
    xb = x_ref[...]
    sb = s_ref[...]
    dn = (((1,), (1,)), ((), ()))
    o = lax.dot_general(xb, wg_ref[...], dn, preferred_element_type=jnp.float32)
    o = o + lax.dot_general(sb, wc_ref[...], dn,
                            preferred_element_type=jnp.float32)
    o = o + b_ref[...]
    o_ref[...] = jnp.where(o > 0, o, jnp.exp(jnp.minimum(o, 0.0)) - 1.0)


def _tc_fuse(x, S, Wg, Wc, bias2d):
    N, D = x.shape
    DO = Wg.shape[0]
    blk = 1000
    grid = (N // blk,)
    return pl.pallas_call(
        _tc_fuse_body,
        grid=grid,
        in_specs=[
            pl.BlockSpec((blk, D), lambda i: (i, 0)),
            pl.BlockSpec((blk, D), lambda i: (i, 0)),
            pl.BlockSpec((DO, D), lambda i: (0, 0)),
            pl.BlockSpec((DO, D), lambda i: (0, 0)),
            pl.BlockSpec((1, DO), lambda i: (0, 0)),
        ],
        out_specs=pl.BlockSpec((blk, DO), lambda i: (i, 0)),
        out_shape=jax.ShapeDtypeStruct((N, DO), jnp.float32),
    )(x, S, Wg, Wc, bias2d)


def kernel(x, W_global, W_local, bias, node_ids, neigh_ids, deg):
    N, D = x.shape
    E = neigh_ids.shape[0]
    deg_static = E // N
    segsum = _make_segsum_sc(N, E, D, deg_static)
    S = segsum(x, neigh_ids.astype(jnp.int32))
    inv_deg = 1.0 / jnp.asarray(deg, jnp.float32)
    Wc = (W_local + W_global) * inv_deg
    out = _tc_fuse(x, S, W_global, Wc, bias.reshape(1, -1))
    return out


# speedup vs baseline: 14.0211x; 1.5543x over previous
import functools

import jax
import jax.numpy as jnp
from jax import lax
from jax.experimental import pallas as pl
from jax.experimental.pallas import tpu as pltpu
from jax.experimental.pallas import tpu_sc as plsc

NUM_CORES = 2
NUM_SUBCORES = 16
NUM_WORKERS = NUM_CORES * NUM_SUBCORES


def _make_segsum_sc(N, E, D, deg):
    n_base = N // NUM_WORKERS
    n_last = N - n_base * (NUM_WORKERS - 1)
    assert n_base % 2 == 0 and n_last % 2 == 0
    max_nodes = n_last

    mesh = plsc.VectorSubcoreMesh(
        core_axis_name="c", subcore_axis_name="s",
        num_cores=NUM_CORES, num_subcores=NUM_SUBCORES)

    @functools.partial(
        pl.kernel,
        out_type=jax.ShapeDtypeStruct((N, D), jnp.float32),
        mesh=mesh,
        scratch_types=[
            pltpu.VMEM((max_nodes * deg,), jnp.int32),
            pltpu.VMEM((deg, D), jnp.float32),
            pltpu.VMEM((deg, D), jnp.float32),
            pltpu.VMEM((deg, D), jnp.float32),
            pltpu.VMEM((deg, D), jnp.float32),
            pltpu.VMEM((max_nodes, D), jnp.float32),
            pltpu.SemaphoreType.DMA,
            pltpu.SemaphoreType.DMA,
            pltpu.SemaphoreType.DMA,
            pltpu.SemaphoreType.DMA,
        ],
    )
    def segsum(x_hbm, nids_hbm, s_hbm, idx_all, buf0, buf1, buf2, buf3,
               out_all, sem0, sem1, sem2, sem3):
        c = lax.axis_index("c")
        s = lax.axis_index("s")
        w = c * NUM_SUBCORES + s
        is_last = w == NUM_WORKERS - 1
        base = w * n_base
        n_w = jnp.where(is_last, n_last, n_base)

        @pl.when(jnp.logical_not(is_last))
        def _():
            pltpu.sync_copy(nids_hbm.at[pl.ds(base * deg, n_base * deg)],
                            idx_all.at[pl.ds(0, n_base * deg)])

        @pl.when(is_last)
        def _():
            pltpu.sync_copy(nids_hbm.at[pl.ds(base * deg, n_last * deg)],
                            idx_all)

        def start_gather(i, buf, sem):
            pltpu.async_copy(
                x_hbm.at[idx_all.at[pl.ds(i * deg, deg)]], buf, sem)

        def wait_gather(buf, sem):
            pltpu.make_async_copy(
                x_hbm.at[idx_all.at[pl.ds(0, deg)]], buf, sem).wait()

        def reduce_into(buf, i):
            for cc in range(D // 16):
                sl = pl.ds(cc * 16, 16)
                accs = [buf[k, sl] for k in range(4)]
                for r in range(4, deg):
                    k = r % 4
                    accs[k] = accs[k] + buf[r, sl]
                out_all[i, sl] = (accs[0] + accs[1]) + (accs[2] + accs[3])

        bufs = (buf0, buf1, buf2, buf3)
        sems = (sem0, sem1, sem2, sem3)
        NBUF = 4
        assert n_base % NBUF == 0 and n_last % NBUF == 0

        for k in range(NBUF - 1):
            start_gather(k, bufs[k], sems[k])

        def body(io, carry):
            i = NBUF * io
            for k in range(NBUF):
                nxt = i + k + NBUF - 1

                @pl.when(nxt < n_w)
                def _(nxt=nxt, k=k):
                    start_gather(nxt, bufs[(k + NBUF - 1) % NBUF],
                                 sems[(k + NBUF - 1) % NBUF])

                wait_gather(bufs[k], sems[k])
                reduce_into(bufs[k], i + k)
            return carry

        lax.fori_loop(0, n_w // NBUF, body, 0)

        @pl.when(jnp.logical_not(is_last))
        def _():
            pltpu.sync_copy(out_all.at[pl.ds(0, n_base)],
                            s_hbm.at[pl.ds(base, n_base)])

        @pl.when(is_last)
        def _():
            pltpu.sync_copy(out_all, s_hbm.at[pl.ds(base, n_last)])

    return segsum


def _tc_fuse_body(x_ref, s_ref, wg_ref, wc_ref, b_ref, o_ref):
    xb = x_ref[...]
    sb = s_ref[...]
    dn = (((1,), (1,)), ((), ()))
    o = lax.dot_general(xb, wg_ref[...], dn, preferred_element_type=jnp.float32)
    o = o + lax.dot_general(sb, wc_ref[...], dn,
                            preferred_element_type=jnp.float32)
    o = o + b_ref[...]
    o_ref[...] = jnp.where(o > 0, o, jnp.exp(jnp.minimum(o, 0.0)) - 1.0)


def _tc_fuse(x, S, Wg, Wc, bias2d):
    N, D = x.shape
    DO = Wg.shape[0]
    blk = 1000
    grid = (N // blk,)
    return pl.pallas_call(
        _tc_fuse_body,
        grid=grid,
        in_specs=[
            pl.BlockSpec((blk, D), lambda i: (i, 0)),
            pl.BlockSpec((blk, D), lambda i: (i, 0)),
            pl.BlockSpec((DO, D), lambda i: (0, 0)),
            pl.BlockSpec((DO, D), lambda i: (0, 0)),
            pl.BlockSpec((1, DO), lambda i: (0, 0)),
        ],
        out_specs=pl.BlockSpec((blk, DO), lambda i: (i, 0)),
        out_shape=jax.ShapeDtypeStruct((N, DO), jnp.float32),
    )(x, S, Wg, Wc, bias2d)


def kernel(x, W_global, W_local, bias, node_ids, neigh_ids, deg):
    N, D = x.shape
    E = neigh_ids.shape[0]
    deg_static = E // N
    segsum = _make_segsum_sc(N, E, D, deg_static)
    S = segsum(x, neigh_ids.astype(jnp.int32))
    inv_deg = 1.0 / jnp.asarray(deg, jnp.float32)
    Wc = (W_local + W_global) * inv_deg
    out = _tc_fuse(x, S, W_global, Wc, bias.reshape(1, -1))
    return out


# speedup vs baseline: 16.2636x; 1.1599x over previous
import functools

import jax
import jax.numpy as jnp
from jax import lax
from jax.experimental import pallas as pl
from jax.experimental.pallas import tpu as pltpu
from jax.experimental.pallas import tpu_sc as plsc

NUM_CORES = 2
NUM_SUBCORES = 16
NUM_WORKERS = NUM_CORES * NUM_SUBCORES
CHUNK = 16


def _make_segsum_sc(N, D, deg):
    n_chunks = N // CHUNK
    assert n_chunks * CHUNK == N
    ch_base = n_chunks // NUM_WORKERS
    n_hi = n_chunks - ch_base * NUM_WORKERS
    ch_hi = ch_base + 1
    max_nodes = ch_hi * CHUNK

    mesh = plsc.VectorSubcoreMesh(
        core_axis_name="c", subcore_axis_name="s",
        num_cores=NUM_CORES, num_subcores=NUM_SUBCORES)

    @functools.partial(
        pl.kernel,
        out_type=jax.ShapeDtypeStruct((N, D), jnp.float32),
        mesh=mesh,
        scratch_types=[
            pltpu.VMEM((ch_hi, deg, CHUNK), jnp.int32),
            pltpu.VMEM((max_nodes, D), jnp.float32),
            pltpu.SemaphoreType.DMA,
            pltpu.SemaphoreType.DMA,
        ],
    )
    def segsum(x_hbm, nt_hbm, s_hbm, idxT, out_all, sem0, sem1):
        c = lax.axis_index("c")
        s = lax.axis_index("s")
        w = c * NUM_SUBCORES + s
        is_hi = w < n_hi
        chunk0 = w * ch_base + jnp.minimum(w, n_hi)
        chunks_w = jnp.where(is_hi, ch_hi, ch_base)
        base = chunk0 * CHUNK

        @pl.when(is_hi)
        def _():
            pltpu.sync_copy(nt_hbm.at[pl.ds(chunk0, ch_hi)], idxT)

        @pl.when(jnp.logical_not(is_hi))
        def _():
            pltpu.sync_copy(nt_hbm.at[pl.ds(chunk0, ch_base)],
                            idxT.at[pl.ds(0, ch_base)])

        zeros_f = jnp.zeros((16,), jnp.float32)

        def zrow(i, carry):
            for cc in range(D // 16):
                out_all[i, pl.ds(cc * 16, 16)] = zeros_f
            return carry

        lax.fori_loop(0, max_nodes, zrow, 0)

        def issue(ch, sem):
            dst = out_all.at[pl.ds(ch * CHUNK, CHUNK)]
            for g in range(deg):
                pltpu.async_copy(x_hbm.at[idxT.at[ch, g]], dst, sem, add=True)

        def drain(ch, sem):
            dst = out_all.at[pl.ds(ch * CHUNK, CHUNK)]
            for g in range(deg):
                pltpu.make_async_copy(x_hbm.at[idxT.at[ch, g]], dst,
                                      sem).wait()

        issue(0, sem0)

        def body(io, carry):
            ch_odd = 2 * io + 1

            @pl.when(ch_odd < chunks_w)
            def _():
                issue(ch_odd, sem1)
                drain(ch_odd - 1, sem0)

            ch_even = 2 * io + 2

            @pl.when(ch_even < chunks_w)
            def _():
                issue(ch_even, sem0)
                drain(ch_even - 1, sem1)

            return carry

        lax.fori_loop(0, chunks_w // 2, body, 0)
        ch_last = chunks_w - 1

        @pl.when(ch_last % 2 == 0)
        def _():
            drain(ch_last, sem0)

        @pl.when(ch_last % 2 == 1)
        def _():
            drain(ch_last, sem1)

        @pl.when(is_hi)
        def _():
            pltpu.sync_copy(out_all, s_hbm.at[pl.ds(base, max_nodes)])

        @pl.when(jnp.logical_not(is_hi))
        def _():
            pltpu.sync_copy(out_all.at[pl.ds(0, ch_base * CHUNK)],
                            s_hbm.at[pl.ds(base, ch_base * CHUNK)])

    return segsum


def _tc_fuse_body(x_ref, s_ref, wg_ref, wc_ref, b_ref, o_ref):
    xb = x_ref[...]
    sb = s_ref[...]
    dn = (((1,), (1,)), ((), ()))
    o = lax.dot_general(xb, wg_ref[...], dn, preferred_element_type=jnp.float32)
    o = o + lax.dot_general(sb, wc_ref[...], dn,
                            preferred_element_type=jnp.float32)
    o = o + b_ref[...]
    o_ref[...] = jnp.where(o > 0, o, jnp.exp(jnp.minimum(o, 0.0)) - 1.0)


def _tc_fuse(x, S, Wg, Wc, bias2d):
    N, D = x.shape
    DO = Wg.shape[0]
    blk = 1000
    grid = (N // blk,)
    return pl.pallas_call(
        _tc_fuse_body,
        grid=grid,
        in_specs=[
            pl.BlockSpec((blk, D), lambda i: (i, 0)),
            pl.BlockSpec((blk, D), lambda i: (i, 0)),
            pl.BlockSpec((DO, D), lambda i: (0, 0)),
            pl.BlockSpec((DO, D), lambda i: (0, 0)),
            pl.BlockSpec((1, DO), lambda i: (0, 0)),
        ],
        out_specs=pl.BlockSpec((blk, DO), lambda i: (i, 0)),
        out_shape=jax.ShapeDtypeStruct((N, DO), jnp.float32),
    )(x, S, Wg, Wc, bias2d)


def kernel(x, W_global, W_local, bias, node_ids, neigh_ids, deg):
    N, D = x.shape
    E = neigh_ids.shape[0]
    deg_static = E // N
    nT = (neigh_ids.astype(jnp.int32)
          .reshape(N // CHUNK, CHUNK, deg_static)
          .transpose(0, 2, 1))
    segsum = _make_segsum_sc(N, D, deg_static)
    S = segsum(x, nT)
    inv_deg = 1.0 / jnp.asarray(deg, jnp.float32)
    Wc = (W_local + W_global) * inv_deg
    out = _tc_fuse(x, S, W_global, Wc, bias.reshape(1, -1))
    return out


# speedup vs baseline: 17.7488x; 1.0913x over previous
import functools

import jax
import jax.numpy as jnp
from jax import lax
from jax.experimental import pallas as pl
from jax.experimental.pallas import tpu as pltpu
from jax.experimental.pallas import tpu_sc as plsc

NUM_CORES = 2
NUM_SUBCORES = 16
NUM_WORKERS = NUM_CORES * NUM_SUBCORES
CHUNK = 80


def _make_segsum_sc(N, D, deg):
    n_chunks = N // CHUNK
    assert n_chunks * CHUNK == N
    ch_base = n_chunks // NUM_WORKERS
    n_hi = n_chunks - ch_base * NUM_WORKERS
    ch_hi = ch_base + 1
    max_nodes = ch_hi * CHUNK

    mesh = plsc.VectorSubcoreMesh(
        core_axis_name="c", subcore_axis_name="s",
        num_cores=NUM_CORES, num_subcores=NUM_SUBCORES)

    @functools.partial(
        pl.kernel,
        out_type=jax.ShapeDtypeStruct((N, D), jnp.float32),
        mesh=mesh,
        scratch_types=[
            pltpu.VMEM((ch_hi, deg, CHUNK), jnp.int32),
            pltpu.VMEM((max_nodes, D), jnp.float32),
            pltpu.SemaphoreType.DMA,
            pltpu.SemaphoreType.DMA,
        ],
    )
    def segsum(x_hbm, nt_hbm, s_hbm, idxT, out_all, sem0, sem1):
        c = lax.axis_index("c")
        s = lax.axis_index("s")
        w = c * NUM_SUBCORES + s
        is_hi = w < n_hi
        chunk0 = w * ch_base + jnp.minimum(w, n_hi)
        chunks_w = jnp.where(is_hi, ch_hi, ch_base)
        base = chunk0 * CHUNK

        @pl.when(is_hi)
        def _():
            pltpu.sync_copy(nt_hbm.at[pl.ds(chunk0, ch_hi)], idxT)

        @pl.when(jnp.logical_not(is_hi))
        def _():
            pltpu.sync_copy(nt_hbm.at[pl.ds(chunk0, ch_base)],
                            idxT.at[pl.ds(0, ch_base)])

        zeros_f = jnp.zeros((16,), jnp.float32)

        def zero_chunk(ch):
            def zrow(i, carry):
                for cc in range(D // 16):
                    out_all[i, pl.ds(cc * 16, 16)] = zeros_f
                return carry

            lax.fori_loop(ch * CHUNK, (ch + 1) * CHUNK, zrow, 0)

        def zero_next(ch):
            @pl.when(ch < chunks_w)
            def _():
                zero_chunk(ch)

        def issue(ch, sem):
            dst = out_all.at[pl.ds(ch * CHUNK, CHUNK)]
            for g in range(deg):
                pltpu.async_copy(x_hbm.at[idxT.at[ch, g]], dst, sem, add=True)

        def drain(ch, sem):
            dst = out_all.at[pl.ds(ch * CHUNK, CHUNK)]
            for g in range(deg):
                pltpu.make_async_copy(x_hbm.at[idxT.at[ch, g]], dst,
                                      sem).wait()

        zero_chunk(0)
        issue(0, sem0)
        zero_next(1)

        def body(io, carry):
            ch_odd = 2 * io + 1

            @pl.when(ch_odd < chunks_w)
            def _():
                issue(ch_odd, sem1)
                zero_next(ch_odd + 1)
                drain(ch_odd - 1, sem0)

            ch_even = 2 * io + 2

            @pl.when(ch_even < chunks_w)
            def _():
                issue(ch_even, sem0)
                zero_next(ch_even + 1)
                drain(ch_even - 1, sem1)

            return carry

        lax.fori_loop(0, chunks_w // 2, body, 0)
        ch_last = chunks_w - 1

        @pl.when(ch_last % 2 == 0)
        def _():
            drain(ch_last, sem0)

        @pl.when(ch_last % 2 == 1)
        def _():
            drain(ch_last, sem1)

        @pl.when(is_hi)
        def _():
            pltpu.sync_copy(out_all, s_hbm.at[pl.ds(base, max_nodes)])

        @pl.when(jnp.logical_not(is_hi))
        def _():
            pltpu.sync_copy(out_all.at[pl.ds(0, ch_base * CHUNK)],
                            s_hbm.at[pl.ds(base, ch_base * CHUNK)])

    return segsum


def _tc_fuse_body(x_ref, s_ref, wg_ref, wc_ref, b_ref, o_ref):
    xb = x_ref[...]
    sb = s_ref[...]
    dn = (((1,), (1,)), ((), ()))
    o = lax.dot_general(xb, wg_ref[...], dn, preferred_element_type=jnp.float32)
    o = o + lax.dot_general(sb, wc_ref[...], dn,
                            preferred_element_type=jnp.float32)
    o = o + b_ref[...]
    o_ref[...] = jnp.where(o > 0, o, jnp.exp(jnp.minimum(o, 0.0)) - 1.0)


def _tc_fuse(x, S, Wg, Wc, bias2d):
    N, D = x.shape
    DO = Wg.shape[0]
    blk = 1000
    grid = (N // blk,)
    return pl.pallas_call(
        _tc_fuse_body,
        grid=grid,
        in_specs=[
            pl.BlockSpec((blk, D), lambda i: (i, 0)),
            pl.BlockSpec((blk, D), lambda i: (i, 0)),
            pl.BlockSpec((DO, D), lambda i: (0, 0)),
            pl.BlockSpec((DO, D), lambda i: (0, 0)),
            pl.BlockSpec((1, DO), lambda i: (0, 0)),
        ],
        out_specs=pl.BlockSpec((blk, DO), lambda i: (i, 0)),
        out_shape=jax.ShapeDtypeStruct((N, DO), jnp.float32),
    )(x, S, Wg, Wc, bias2d)


def kernel(x, W_global, W_local, bias, node_ids, neigh_ids, deg):
    N, D = x.shape
    E = neigh_ids.shape[0]
    deg_static = E // N
    nT = (neigh_ids.astype(jnp.int32)
          .reshape(N // CHUNK, CHUNK, deg_static)
          .transpose(0, 2, 1))
    segsum = _make_segsum_sc(N, D, deg_static)
    S = segsum(x, nT)
    inv_deg = 1.0 / jnp.asarray(deg, jnp.float32)
    Wc = (W_local + W_global) * inv_deg
    out = _tc_fuse(x, S, W_global, Wc, bias.reshape(1, -1))
    return out
